# Initial kernel scaffold; baseline (speedup 1.0000x reference)
import jax
import jax.numpy as jnp
from jax.experimental import pallas as pl


def kernel(x, switch_w, switch_b, fn1_w, fn1_b, fn2_w, fn2_b, fn3_w, fn3_b):
    raise NotImplementedError("write your pallas kernel here")


# speedup vs baseline: 1.1367x; 1.1367x over previous
import functools

import jax
import jax.numpy as jnp
from jax.experimental import pallas as pl
from jax.experimental.pallas import tpu as pltpu

D = 1024
CH = 1024
E = 8
HID = 4096
TM = 1024
HC = 512


def _moe_kernel(x_ref, sw_ref, f1_ref, f3_ref, f2_ref, out_ref, wall_ref):
    e = pl.program_id(1)
    h = pl.program_id(2)

    @pl.when((e == 0) & (h == 0))
    def _gate():
        logits = jax.lax.dot_general(
            x_ref[...].astype(jnp.bfloat16), sw_ref[...].astype(jnp.bfloat16),
            (((1,), (1,)), ((), ())),
            preferred_element_type=jnp.float32)
        idx = jax.lax.broadcasted_iota(jnp.int32, logits.shape, 1)
        m1 = jnp.max(logits, axis=1, keepdims=True)
        i1 = jnp.min(jnp.where(logits == m1, idx, E), axis=1, keepdims=True)
        mask1 = idx == i1
        l2 = jnp.where(mask1, -jnp.inf, logits)
        m2 = jnp.max(l2, axis=1, keepdims=True)
        i2 = jnp.min(jnp.where(l2 == m2, idx, E), axis=1, keepdims=True)
        w1 = 1.0 / (1.0 + jnp.exp(m2 - m1))
        w2 = 1.0 - w1
        wall_ref[...] = jnp.where(idx == i1, w1, 0.0) + jnp.where(idx == i2, w2, 0.0)
        out_ref[...] = jnp.zeros_like(out_ref)

    xb = x_ref[...].astype(jnp.bfloat16)
    f1 = f1_ref[0]
    f3 = f3_ref[0]
    f2 = f2_ref[0]
    h1 = jax.lax.dot_general(xb, f1, (((1,), (1,)), ((), ())),
                             preferred_element_type=jnp.float32)
    h3 = jax.lax.dot_general(xb, f3, (((1,), (1,)), ((), ())),
                             preferred_element_type=jnp.float32)
    g = jax.nn.silu(h1) * h3
    wa = wall_ref[...]
    col = jax.lax.broadcasted_iota(jnp.int32, wa.shape, 1)
    w_col = jnp.sum(jnp.where(col == e, wa, 0.0), axis=1, keepdims=True)
    gb = (g * w_col).astype(jnp.bfloat16)
    out_ref[...] += jax.lax.dot_general(gb, f2, (((1,), (1,)), ((), ())),
                                        preferred_element_type=jnp.float32)


@functools.partial(jax.jit, static_argnames=())
def kernel(x, switch_w, switch_b, fn1_w, fn1_b, fn2_w, fn2_b, fn3_w, fn3_b):
    B, N, d = x.shape
    xf = x.reshape(-1, d)
    T = xf.shape[0]
    f1b = fn1_w.astype(jnp.bfloat16)
    f3b = fn3_w.astype(jnp.bfloat16)
    f2b = fn2_w.astype(jnp.bfloat16)
    out = pl.pallas_call(
        _moe_kernel,
        grid=(T // TM, E, HID // HC),
        in_specs=[
            pl.BlockSpec((TM, D), lambda t, e, h: (t, 0)),
            pl.BlockSpec((E, D), lambda t, e, h: (0, 0)),
            pl.BlockSpec((1, HC, D), lambda t, e, h: (e, h, 0)),
            pl.BlockSpec((1, HC, D), lambda t, e, h: (e, h, 0)),
            pl.BlockSpec((1, CH, HC), lambda t, e, h: (e, 0, h)),
        ],
        out_specs=pl.BlockSpec((TM, CH), lambda t, e, h: (t, 0)),
        out_shape=jax.ShapeDtypeStruct((T, CH), jnp.float32),
        scratch_shapes=[pltpu.VMEM((TM, E), jnp.float32)],
        compiler_params=pltpu.CompilerParams(
            dimension_semantics=("arbitrary", "arbitrary", "arbitrary")),
    )(xf, switch_w, f1b, f3b, f2b)
    return out.reshape(B, N, CH)


# speedup vs baseline: 1.2376x; 1.0888x over previous
import functools

import jax
import jax.numpy as jnp
from jax import lax
from jax.experimental import pallas as pl
from jax.experimental.pallas import tpu as pltpu
from jax.experimental.pallas import tpu_sc as plsc

D = 1024
CH = 1024
E = 8
HID = 4096
TOPK = 2
T = 4096
M = 256
NT = T * TOPK // M + E
P = NT * M
HC = 512
NH = HID // HC

_SC_NC = 2
_SC_NS = 16
_NW = _SC_NC * _SC_NS


def _gate_kernel(x_ref, sw_ref, sel_ref, w_ref):
    logits = lax.dot_general(
        x_ref[...].astype(jnp.bfloat16), sw_ref[...].astype(jnp.bfloat16),
        (((1,), (1,)), ((), ())), preferred_element_type=jnp.float32)
    idx = lax.broadcasted_iota(jnp.int32, logits.shape, 1)
    m1 = jnp.max(logits, axis=1, keepdims=True)
    i1 = jnp.min(jnp.where(logits == m1, idx, E), axis=1, keepdims=True)
    l2 = jnp.where(idx == i1, -jnp.inf, logits)
    m2 = jnp.max(l2, axis=1, keepdims=True)
    i2 = jnp.min(jnp.where(l2 == m2, idx, E), axis=1, keepdims=True)
    w1 = 1.0 / (1.0 + jnp.exp(m2 - m1))
    col = lax.broadcasted_iota(jnp.int32, (x_ref.shape[0], TOPK), 1)
    sel_ref[...] = jnp.where(col == 0, i1, i2)
    w_ref[...] = jnp.where(col == 0, w1, 1.0 - w1)


def _ffn_kernel(te_ref, xg_ref, w_ref, f1_ref, f3_ref, f2_ref, y_ref):
    h = pl.program_id(1)
    xb = xg_ref[...].astype(jnp.bfloat16)
    h1 = lax.dot_general(xb, f1_ref[0], (((1,), (1,)), ((), ())),
                         preferred_element_type=jnp.float32)
    h3 = lax.dot_general(xb, f3_ref[0], (((1,), (1,)), ((), ())),
                         preferred_element_type=jnp.float32)
    g = jax.nn.silu(h1) * h3 * w_ref[...]

    @pl.when(h == 0)
    def _():
        y_ref[...] = jnp.zeros_like(y_ref)

    y_ref[...] += lax.dot_general(g.astype(jnp.bfloat16), f2_ref[0],
                                  (((1,), (1,)), ((), ())),
                                  preferred_element_type=jnp.float32)


def _add_kernel(a_ref, b_ref, o_ref):
    o_ref[...] = a_ref[...] + b_ref[...]


def _make_sc_gather(n_rows):
    b_per_w = n_rows // _NW
    n_ch = b_per_w // 16
    mesh = plsc.VectorSubcoreMesh(core_axis_name="c", subcore_axis_name="s")

    def body(src_hbm, idx_hbm, out_hbm, idx_v, buf0, buf1, sem0, sem1):
        wid = lax.axis_index("s") * _SC_NC + lax.axis_index("c")
        base = wid * b_per_w
        pltpu.sync_copy(idx_hbm.at[pl.ds(base, b_per_w)], idx_v)
        bufs = (buf0, buf1)
        sems = (sem0, sem1)
        prev = None
        for c in range(n_ch):
            p = c & 1
            idx_reg = idx_v[pl.ds(c * 16, 16)]
            cp = pltpu.async_copy(src_hbm.at[idx_reg], bufs[p], sems[p])
            if prev is not None:
                pcp, pc = prev
                pcp.wait()
                pltpu.sync_copy(bufs[pc & 1],
                                out_hbm.at[pl.ds(base + pc * 16, 16)])
            prev = (cp, c)
        pcp, pc = prev
        pcp.wait()
        pltpu.sync_copy(bufs[pc & 1], out_hbm.at[pl.ds(base + pc * 16, 16)])

    return functools.partial(
        pl.kernel, body, mesh=mesh,
        out_type=jax.ShapeDtypeStruct((n_rows, D), jnp.float32),
        scratch_types=[
            pltpu.VMEM((b_per_w,), jnp.int32),
            pltpu.VMEM((16, D), jnp.float32),
            pltpu.VMEM((16, D), jnp.float32),
            pltpu.SemaphoreType.DMA,
            pltpu.SemaphoreType.DMA,
        ])()


def kernel(x, switch_w, switch_b, fn1_w, fn1_b, fn2_w, fn2_b, fn3_w, fn3_b):
    B, N, d = x.shape
    xf = x.reshape(-1, d)

    sel, w2 = pl.pallas_call(
        _gate_kernel,
        in_specs=[pl.BlockSpec((T, D), lambda: (0, 0)),
                  pl.BlockSpec((E, D), lambda: (0, 0))],
        out_specs=[pl.BlockSpec((T, TOPK), lambda: (0, 0)),
                   pl.BlockSpec((T, TOPK), lambda: (0, 0))],
        out_shape=[jax.ShapeDtypeStruct((T, TOPK), jnp.int32),
                   jax.ShapeDtypeStruct((T, TOPK), jnp.float32)],
    )(xf, switch_w)

    e_flat = sel.reshape(-1)
    w_flat = w2.reshape(-1)
    tok_flat = jnp.arange(T * TOPK, dtype=jnp.int32) // TOPK
    onehot = (e_flat[:, None] == jnp.arange(E, dtype=jnp.int32)).astype(jnp.int32)
    counts = onehot.sum(axis=0)
    rank = jnp.take_along_axis(jnp.cumsum(onehot, axis=0) - onehot,
                               e_flat[:, None], axis=1)[:, 0]
    pad_cnt = ((counts + M - 1) // M) * M
    cum_pad = jnp.cumsum(pad_cnt)
    pad_start = cum_pad - pad_cnt
    pos = (pad_start[e_flat] + rank).astype(jnp.int32)
    tok_slot = jnp.zeros((P,), jnp.int32).at[pos].set(tok_flat)
    w_slot = jnp.zeros((P,), jnp.float32).at[pos].set(w_flat)
    tile_start = jnp.arange(NT, dtype=jnp.int32) * M
    tile_expert = jnp.minimum(
        (tile_start[:, None] >= cum_pad[None, :]).sum(axis=1), E - 1
    ).astype(jnp.int32)

    xg = _make_sc_gather(P)(xf, tok_slot)

    f1b = fn1_w.astype(jnp.bfloat16)
    f3b = fn3_w.astype(jnp.bfloat16)
    f2b = fn2_w.astype(jnp.bfloat16)
    y = pl.pallas_call(
        _ffn_kernel,
        grid_spec=pltpu.PrefetchScalarGridSpec(
            num_scalar_prefetch=1,
            grid=(NT, NH),
            in_specs=[
                pl.BlockSpec((M, D), lambda t, h, te: (t, 0)),
                pl.BlockSpec((M, 1), lambda t, h, te: (t, 0)),
                pl.BlockSpec((1, HC, D), lambda t, h, te: (te[t], h, 0)),
                pl.BlockSpec((1, HC, D), lambda t, h, te: (te[t], h, 0)),
                pl.BlockSpec((1, CH, HC), lambda t, h, te: (te[t], 0, h)),
            ],
            out_specs=pl.BlockSpec((M, CH), lambda t, h, te: (t, 0)),
        ),
        out_shape=jax.ShapeDtypeStruct((P, CH), jnp.float32),
        compiler_params=pltpu.CompilerParams(
            dimension_semantics=("arbitrary", "arbitrary")),
    )(tile_expert, xg, w_slot[:, None], f1b, f3b, f2b)

    slot_ab = pos.reshape(T, TOPK).T.reshape(-1)
    y_ab = _make_sc_gather(2 * T)(y, slot_ab)
    out = pl.pallas_call(
        _add_kernel,
        grid=(4,),
        in_specs=[pl.BlockSpec((T // 4, CH), lambda i: (i, 0)),
                  pl.BlockSpec((T // 4, CH), lambda i: (i + 4, 0))],
        out_specs=pl.BlockSpec((T // 4, CH), lambda i: (i, 0)),
        out_shape=jax.ShapeDtypeStruct((T, CH), jnp.float32),
    )(y_ab, y_ab)
    return out.reshape(B, N, CH)
